# Initial kernel scaffold; baseline (speedup 1.0000x reference)
import jax
import jax.numpy as jnp
from jax.experimental import pallas as pl


def kernel(x, emb):
    raise NotImplementedError("write your pallas kernel here")


# speedup vs baseline: 2.5178x; 2.5178x over previous
import jax
import jax.numpy as jnp
from jax.experimental import pallas as pl


def _copy_body(e_ref, o_ref):
    o_ref[...] = e_ref[...]


def kernel(x, emb):
    seq_len = x.shape[1]
    d_model = emb.shape[1]
    src = emb[:seq_len]
    block = 512
    grid = (seq_len // block,)
    out = pl.pallas_call(
        _copy_body,
        grid=grid,
        in_specs=[pl.BlockSpec((block, d_model), lambda i: (i, 0))],
        out_specs=pl.BlockSpec((block, d_model), lambda i: (i, 0)),
        out_shape=jax.ShapeDtypeStruct((seq_len, d_model), emb.dtype),
    )(src)
    return out[None]
